# baseline (speedup 1.0000x reference)
import functools

import jax
import jax.numpy as jnp
from jax import lax
from jax.experimental import pallas as pl
from jax.experimental.pallas import tpu as pltpu
from jax.experimental.pallas import tpu_sc as plsc

N_NODES = 10000
N_EDGES = 160000
D = 128
NPAD = 10240

NC, NS = 2, 16
NW = NC * NS
CH = 128
NCHUNK = N_EDGES // CH
RPT = NPAD // NS

def _worker_id():
    return lax.axis_index("s") * NC + lax.axis_index("c")


def _num_chunks(wid):
    base = NCHUNK // NW
    return base + (wid < (NCHUNK - base * NW)).astype(jnp.int32)


DP = D // 2


def _gather_body(h_hbm, dst_hbm, src_hbm, hd_hbm, hs_hbm,
                 idx_d, rows_d, idx_s, rows_s, sem_d, sem_s):
    wid = _worker_id()
    nk = _num_chunks(wid)

    def step(j, carry):
        off = (j * NW + wid) * CH
        pltpu.sync_copy(dst_hbm.at[pl.ds(off, CH)], idx_d)
        cp_d = pltpu.async_copy(h_hbm.at[idx_d], rows_d, sem_d)
        pltpu.sync_copy(src_hbm.at[pl.ds(off, CH)], idx_s)
        cp_s = pltpu.async_copy(h_hbm.at[idx_s], rows_s, sem_s)
        cp_d.wait()
        pltpu.sync_copy(rows_d, hd_hbm.at[pl.ds(off, CH)])
        cp_s.wait()
        pltpu.sync_copy(rows_s, hs_hbm.at[pl.ds(off, CH)])
        return carry

    lax.fori_loop(0, nk, step, 0)


@functools.lru_cache(maxsize=None)
def _build_gather():
    return pl.kernel(
        _gather_body,
        out_type=[jax.ShapeDtypeStruct((N_EDGES, DP), jnp.int32),
                  jax.ShapeDtypeStruct((N_EDGES, DP), jnp.int32)],
        mesh=plsc.VectorSubcoreMesh(core_axis_name="c", subcore_axis_name="s"),
        scratch_types=[pltpu.VMEM((CH,), jnp.int32),
                       pltpu.VMEM((CH, DP), jnp.int32),
                       pltpu.VMEM((CH,), jnp.int32),
                       pltpu.VMEM((CH, DP), jnp.int32),
                       pltpu.SemaphoreType.DMA,
                       pltpu.SemaphoreType.DMA],
        compiler_params=pltpu.CompilerParams(use_tc_tiling_on_sc=False),
    )


def _gather(h, dst, src):
    return _build_gather()(h, dst, src)


def _scatter_body(msg_hbm, dst_hbm, zeros_hbm, out_hbm, idx_v, rows_v, acc_sh):
    c = lax.axis_index("c")
    s = lax.axis_index("s")
    wid = _worker_id()
    nk = _num_chunks(wid)

    pltpu.sync_copy(zeros_hbm.at[pl.ds(s * RPT, RPT)],
                    acc_sh.at[pl.ds(s * RPT, RPT)])
    plsc.subcore_barrier()

    def step(j, carry):
        off = (j * NW + wid) * CH
        pltpu.sync_copy(msg_hbm.at[pl.ds(off, CH)], rows_v)
        pltpu.sync_copy(dst_hbm.at[pl.ds(off, CH)], idx_v)
        pltpu.sync_copy(rows_v, acc_sh.at[idx_v], add=True)
        return carry

    lax.fori_loop(0, nk, step, 0)
    plsc.subcore_barrier()
    pltpu.sync_copy(acc_sh.at[pl.ds(s * RPT, RPT)],
                    out_hbm.at[c, pl.ds(s * RPT, RPT)])


@functools.lru_cache(maxsize=None)
def _build_scatter():
    return pl.kernel(
        _scatter_body,
        out_type=[jax.ShapeDtypeStruct((NC, NPAD, D), jnp.float32)],
        mesh=plsc.VectorSubcoreMesh(core_axis_name="c", subcore_axis_name="s"),
        scratch_types=[pltpu.VMEM((CH,), jnp.int32),
                       pltpu.VMEM((CH, D), jnp.float32),
                       pltpu.VMEM_SHARED((NPAD, D), jnp.float32)],
    )


def _scatter(msg, dst, zeros_pad):
    return _build_scatter()(msg, dst, zeros_pad)


BE = 1280


def _mlp_body(hd_ref, hs_ref, ea_ref, ax_ref, W1_ref, b1_ref, W2_ref, b2_ref,
              out_ref):
    hd = hd_ref[...]
    hs = hs_ref[...]
    acc = jnp.dot(hd, W1_ref[0:D, :], preferred_element_type=jnp.float32)
    acc += jnp.dot(hs, W1_ref[D:2 * D, :], preferred_element_type=jnp.float32)
    acc += jnp.dot(ea_ref[...], W1_ref[2 * D:3 * D, :],
                   preferred_element_type=jnp.float32)
    acc += jnp.dot(ax_ref[...], W1_ref[3 * D:, :],
                   preferred_element_type=jnp.float32)
    acc += b1_ref[...]
    hdn = jnp.maximum(acc, 0.0).astype(jnp.bfloat16)
    out_ref[...] = jnp.dot(hdn, W2_ref[...],
                           preferred_element_type=jnp.float32) + b2_ref[...]


def _edge_mlp(hd, hs, ea, ax, W1r, b1r, W2r, b2r):
    n_in = 2 * D + ea.shape[1] + ax.shape[1]
    hid = W1r.shape[1]
    grid = N_EDGES // BE
    return pl.pallas_call(
        _mlp_body,
        grid=(grid,),
        in_specs=[
            pl.BlockSpec((BE, D), lambda i: (i, 0)),
            pl.BlockSpec((BE, D), lambda i: (i, 0)),
            pl.BlockSpec((BE, ea.shape[1]), lambda i: (i, 0)),
            pl.BlockSpec((BE, ax.shape[1]), lambda i: (i, 0)),
            pl.BlockSpec((n_in, hid), lambda i: (0, 0)),
            pl.BlockSpec((1, hid), lambda i: (0, 0)),
            pl.BlockSpec((hid, D), lambda i: (0, 0)),
            pl.BlockSpec((1, D), lambda i: (0, 0)),
        ],
        out_specs=pl.BlockSpec((BE, D), lambda i: (i, 0)),
        out_shape=jax.ShapeDtypeStruct((N_EDGES, D), jnp.float32),
    )(hd, hs, ea, ax, W1r, b1r, W2r, b2r)


BN = 2000


def _gru_body(ap_ref, h_ref, Wih_ref, bih_ref, Whh_ref, bhh_ref, out_ref):
    a = ap_ref[0] + ap_ref[1]
    h = h_ref[...]
    gi = lax.dot_general(a, Wih_ref[...], (((1,), (1,)), ((), ())),
                         preferred_element_type=jnp.float32) + bih_ref[...]
    gh = lax.dot_general(h, Whh_ref[...], (((1,), (1,)), ((), ())),
                         preferred_element_type=jnp.float32) + bhh_ref[...]
    r = jax.nn.sigmoid(gi[:, 0:D] + gh[:, 0:D])
    z = jax.nn.sigmoid(gi[:, D:2 * D] + gh[:, D:2 * D])
    n = jnp.tanh(gi[:, 2 * D:3 * D] + r * gh[:, 2 * D:3 * D])
    out_ref[...] = (1.0 - z) * n + z * h


def _gru(ap, h, Wihr, bihr, Whhr, bhhr):
    grid = N_NODES // BN
    return pl.pallas_call(
        _gru_body,
        grid=(grid,),
        in_specs=[
            pl.BlockSpec((NC, BN, D), lambda i: (0, i, 0)),
            pl.BlockSpec((BN, D), lambda i: (i, 0)),
            pl.BlockSpec((3 * D, D), lambda i: (0, 0)),
            pl.BlockSpec((1, 3 * D), lambda i: (0, 0)),
            pl.BlockSpec((3 * D, D), lambda i: (0, 0)),
            pl.BlockSpec((1, 3 * D), lambda i: (0, 0)),
        ],
        out_specs=pl.BlockSpec((BN, D), lambda i: (i, 0)),
        out_shape=jax.ShapeDtypeStruct((N_NODES, D), jnp.float32),
    )(ap, h, Wihr, bihr, Whhr, bhhr)


def kernel(x, edge_index, edge_attr, auxiliary, W1, b1, W2, b2,
           W_ih, b_ih, W_hh, b_hh):
    ei = edge_index.astype(jnp.int32)
    src = ei[0]
    dst = ei[1]
    zeros_pad = jnp.zeros((NPAD, D), jnp.float32)
    ea_bf = edge_attr.astype(jnp.bfloat16)
    ax_bf = auxiliary.astype(jnp.bfloat16)
    W1_bf = W1.astype(jnp.bfloat16)
    W2_bf = W2.astype(jnp.bfloat16)
    h = x
    for r in range(W1.shape[0]):
        h_pack = lax.bitcast_convert_type(
            h.astype(jnp.bfloat16).reshape(N_NODES, DP, 2), jnp.int32)
        hd_p, hs_p = _gather(h_pack, dst, src)
        hd = lax.bitcast_convert_type(hd_p, jnp.bfloat16).reshape(N_EDGES, D)
        hs = lax.bitcast_convert_type(hs_p, jnp.bfloat16).reshape(N_EDGES, D)
        msg = _edge_mlp(hd, hs, ea_bf, ax_bf,
                        W1_bf[r], b1[r].reshape(1, -1),
                        W2_bf[r], b2[r].reshape(1, -1))
        (ap,) = _scatter(msg, dst, zeros_pad)
        h = _gru(ap, h, W_ih[r], b_ih[r].reshape(1, -1),
                 W_hh[r], b_hh[r].reshape(1, -1))
    return h


# speedup vs baseline: 2.0965x; 2.0965x over previous
import functools

import jax
import jax.numpy as jnp
from jax import lax
from jax.experimental import pallas as pl
from jax.experimental.pallas import tpu as pltpu
from jax.experimental.pallas import tpu_sc as plsc

N_NODES = 10000
N_EDGES = 160000
D = 128
NPAD = 10240

NC, NS = 2, 16
NW = NC * NS
CH = 128
NCHUNK = N_EDGES // CH
RPT = NPAD // NS

def _worker_id():
    return lax.axis_index("s") * NC + lax.axis_index("c")


def _num_chunks(wid):
    base = NCHUNK // NW
    return base + (wid < (NCHUNK - base * NW)).astype(jnp.int32)


DP = D // 2


def _gather_body(h_hbm, dst_hbm, src_hbm, hd_hbm, hs_hbm,
                 idx_d, rows_d, idx_s, rows_s, sem_d, sem_s):
    wid = _worker_id()
    nk = _num_chunks(wid)

    def step(j, carry):
        off = (j * NW + wid) * CH
        pltpu.sync_copy(dst_hbm.at[pl.ds(off, CH)], idx_d)
        cp_d = pltpu.async_copy(h_hbm.at[idx_d], rows_d, sem_d)
        pltpu.sync_copy(src_hbm.at[pl.ds(off, CH)], idx_s)
        cp_s = pltpu.async_copy(h_hbm.at[idx_s], rows_s, sem_s)
        cp_d.wait()
        pltpu.sync_copy(rows_d, hd_hbm.at[pl.ds(off, CH)])
        cp_s.wait()
        pltpu.sync_copy(rows_s, hs_hbm.at[pl.ds(off, CH)])
        return carry

    lax.fori_loop(0, nk, step, 0)


@functools.lru_cache(maxsize=None)
def _build_gather():
    return pl.kernel(
        _gather_body,
        out_type=[jax.ShapeDtypeStruct((N_EDGES, D), jnp.float32),
                  jax.ShapeDtypeStruct((N_EDGES, D), jnp.float32)],
        mesh=plsc.VectorSubcoreMesh(core_axis_name="c", subcore_axis_name="s"),
        scratch_types=[pltpu.VMEM((CH,), jnp.int32),
                       pltpu.VMEM((CH, D), jnp.float32),
                       pltpu.VMEM((CH,), jnp.int32),
                       pltpu.VMEM((CH, D), jnp.float32),
                       pltpu.SemaphoreType.DMA,
                       pltpu.SemaphoreType.DMA],
    )


def _gather(h, dst, src):
    return _build_gather()(h, dst, src)


def _scatter_body(msg_hbm, dst_hbm, zeros_hbm, out_hbm, idx_v, rows_v, acc_sh):
    c = lax.axis_index("c")
    s = lax.axis_index("s")
    wid = _worker_id()
    nk = _num_chunks(wid)

    pltpu.sync_copy(zeros_hbm.at[pl.ds(s * RPT, RPT)],
                    acc_sh.at[pl.ds(s * RPT, RPT)])
    plsc.subcore_barrier()

    def step(j, carry):
        off = (j * NW + wid) * CH
        pltpu.sync_copy(msg_hbm.at[pl.ds(off, CH)], rows_v)
        pltpu.sync_copy(dst_hbm.at[pl.ds(off, CH)], idx_v)
        pltpu.sync_copy(rows_v, acc_sh.at[idx_v], add=True)
        return carry

    lax.fori_loop(0, nk, step, 0)
    plsc.subcore_barrier()
    pltpu.sync_copy(acc_sh.at[pl.ds(s * RPT, RPT)],
                    out_hbm.at[c, pl.ds(s * RPT, RPT)])


@functools.lru_cache(maxsize=None)
def _build_scatter():
    return pl.kernel(
        _scatter_body,
        out_type=[jax.ShapeDtypeStruct((NC, NPAD, D), jnp.float32)],
        mesh=plsc.VectorSubcoreMesh(core_axis_name="c", subcore_axis_name="s"),
        scratch_types=[pltpu.VMEM((CH,), jnp.int32),
                       pltpu.VMEM((CH, D), jnp.float32),
                       pltpu.VMEM_SHARED((NPAD, D), jnp.float32)],
    )


def _scatter(msg, dst, zeros_pad):
    return _build_scatter()(msg, dst, zeros_pad)


BE = 1280


def _mlp_body(hd_ref, hs_ref, ea_ref, ax_ref, W1_ref, b1_ref, W2_ref, b2_ref,
              out_ref):
    hd = hd_ref[...].astype(jnp.bfloat16)
    hs = hs_ref[...].astype(jnp.bfloat16)
    acc = jnp.dot(hd, W1_ref[0:D, :], preferred_element_type=jnp.float32)
    acc += jnp.dot(hs, W1_ref[D:2 * D, :], preferred_element_type=jnp.float32)
    acc += jnp.dot(ea_ref[...], W1_ref[2 * D:3 * D, :],
                   preferred_element_type=jnp.float32)
    acc += jnp.dot(ax_ref[...], W1_ref[3 * D:, :],
                   preferred_element_type=jnp.float32)
    acc += b1_ref[...]
    hdn = jnp.maximum(acc, 0.0).astype(jnp.bfloat16)
    out_ref[...] = jnp.dot(hdn, W2_ref[...],
                           preferred_element_type=jnp.float32) + b2_ref[...]


def _edge_mlp(hd, hs, ea, ax, W1r, b1r, W2r, b2r):
    n_in = 2 * D + ea.shape[1] + ax.shape[1]
    hid = W1r.shape[1]
    grid = N_EDGES // BE
    return pl.pallas_call(
        _mlp_body,
        grid=(grid,),
        in_specs=[
            pl.BlockSpec((BE, D), lambda i: (i, 0)),
            pl.BlockSpec((BE, D), lambda i: (i, 0)),
            pl.BlockSpec((BE, ea.shape[1]), lambda i: (i, 0)),
            pl.BlockSpec((BE, ax.shape[1]), lambda i: (i, 0)),
            pl.BlockSpec((n_in, hid), lambda i: (0, 0)),
            pl.BlockSpec((1, hid), lambda i: (0, 0)),
            pl.BlockSpec((hid, D), lambda i: (0, 0)),
            pl.BlockSpec((1, D), lambda i: (0, 0)),
        ],
        out_specs=pl.BlockSpec((BE, D), lambda i: (i, 0)),
        out_shape=jax.ShapeDtypeStruct((N_EDGES, D), jnp.float32),
    )(hd, hs, ea, ax, W1r, b1r, W2r, b2r)


BN = 2000


def _gru_body(ap_ref, h_ref, Wih_ref, bih_ref, Whh_ref, bhh_ref, out_ref):
    a = ap_ref[0] + ap_ref[1]
    h = h_ref[...]
    gi = lax.dot_general(a, Wih_ref[...], (((1,), (1,)), ((), ())),
                         preferred_element_type=jnp.float32) + bih_ref[...]
    gh = lax.dot_general(h, Whh_ref[...], (((1,), (1,)), ((), ())),
                         preferred_element_type=jnp.float32) + bhh_ref[...]
    r = jax.nn.sigmoid(gi[:, 0:D] + gh[:, 0:D])
    z = jax.nn.sigmoid(gi[:, D:2 * D] + gh[:, D:2 * D])
    n = jnp.tanh(gi[:, 2 * D:3 * D] + r * gh[:, 2 * D:3 * D])
    out_ref[...] = (1.0 - z) * n + z * h


def _gru(ap, h, Wihr, bihr, Whhr, bhhr):
    grid = N_NODES // BN
    return pl.pallas_call(
        _gru_body,
        grid=(grid,),
        in_specs=[
            pl.BlockSpec((NC, BN, D), lambda i: (0, i, 0)),
            pl.BlockSpec((BN, D), lambda i: (i, 0)),
            pl.BlockSpec((3 * D, D), lambda i: (0, 0)),
            pl.BlockSpec((1, 3 * D), lambda i: (0, 0)),
            pl.BlockSpec((3 * D, D), lambda i: (0, 0)),
            pl.BlockSpec((1, 3 * D), lambda i: (0, 0)),
        ],
        out_specs=pl.BlockSpec((BN, D), lambda i: (i, 0)),
        out_shape=jax.ShapeDtypeStruct((N_NODES, D), jnp.float32),
    )(ap, h, Wihr, bihr, Whhr, bhhr)


def kernel(x, edge_index, edge_attr, auxiliary, W1, b1, W2, b2,
           W_ih, b_ih, W_hh, b_hh):
    ei = edge_index.astype(jnp.int32)
    src = ei[0]
    dst = ei[1]
    zeros_pad = jnp.zeros((NPAD, D), jnp.float32)
    ea_bf = edge_attr.astype(jnp.bfloat16)
    ax_bf = auxiliary.astype(jnp.bfloat16)
    W1_bf = W1.astype(jnp.bfloat16)
    W2_bf = W2.astype(jnp.bfloat16)
    h = x
    for r in range(W1.shape[0]):
        hd, hs = _gather(h, dst, src)
        msg = _edge_mlp(hd, hs, ea_bf, ax_bf,
                        W1_bf[r], b1[r].reshape(1, -1),
                        W2_bf[r], b2[r].reshape(1, -1))
        (ap,) = _scatter(msg, dst, zeros_pad)
        h = _gru(ap, h, W_ih[r], b_ih[r].reshape(1, -1),
                 W_hh[r], b_hh[r].reshape(1, -1))
    return h


# speedup vs baseline: 2.6523x; 1.2651x over previous
import functools

import jax
import jax.numpy as jnp
from jax import lax
from jax.experimental import pallas as pl
from jax.experimental.pallas import tpu as pltpu
from jax.experimental.pallas import tpu_sc as plsc

N_NODES = 10000
N_EDGES = 160000
D = 128
NPAD = 10240

NC, NS = 2, 16
NW = NC * NS
CH = 128
NCHUNK = N_EDGES // CH
RPT = NPAD // NS

def _worker_id():
    return lax.axis_index("s") * NC + lax.axis_index("c")


def _num_chunks(wid):
    base = NCHUNK // NW
    return base + (wid < (NCHUNK - base * NW)).astype(jnp.int32)


DP = D // 2


def _gather_body(h_hbm, dst_hbm, src_hbm, hd_hbm, hs_hbm,
                 idx_d, rows_d, idx_s, rows_s, sem_d, sem_s):
    wid = _worker_id()
    nk = _num_chunks(wid)

    def step(j, carry):
        off = (j * NW + wid) * CH
        pltpu.sync_copy(dst_hbm.at[pl.ds(off, CH)], idx_d)
        cp_d = pltpu.async_copy(h_hbm.at[idx_d], rows_d, sem_d)
        pltpu.sync_copy(src_hbm.at[pl.ds(off, CH)], idx_s)
        cp_s = pltpu.async_copy(h_hbm.at[idx_s], rows_s, sem_s)
        cp_d.wait()
        pltpu.sync_copy(rows_d, hd_hbm.at[pl.ds(off, CH)])
        cp_s.wait()
        pltpu.sync_copy(rows_s, hs_hbm.at[pl.ds(off, CH)])
        return carry

    lax.fori_loop(0, nk, step, 0)


@functools.lru_cache(maxsize=None)
def _build_gather():
    return pl.kernel(
        _gather_body,
        out_type=[jax.ShapeDtypeStruct((N_EDGES, D), jnp.float32),
                  jax.ShapeDtypeStruct((N_EDGES, D), jnp.float32)],
        mesh=plsc.VectorSubcoreMesh(core_axis_name="c", subcore_axis_name="s"),
        scratch_types=[pltpu.VMEM((CH,), jnp.int32),
                       pltpu.VMEM((CH, D), jnp.float32),
                       pltpu.VMEM((CH,), jnp.int32),
                       pltpu.VMEM((CH, D), jnp.float32),
                       pltpu.SemaphoreType.DMA,
                       pltpu.SemaphoreType.DMA],
    )


def _gather(h, dst, src):
    return _build_gather()(h, dst, src)


def _scatter_body(msg_hbm, dst_hbm, zeros_hbm, out_hbm, idx_v, rows_v, acc_sh):
    c = lax.axis_index("c")
    s = lax.axis_index("s")
    wid = _worker_id()
    nk = _num_chunks(wid)

    pltpu.sync_copy(zeros_hbm.at[pl.ds(s * RPT, RPT)],
                    acc_sh.at[pl.ds(s * RPT, RPT)])
    plsc.subcore_barrier()

    def step(j, carry):
        off = (j * NW + wid) * CH
        pltpu.sync_copy(msg_hbm.at[pl.ds(off, CH)], rows_v)
        pltpu.sync_copy(dst_hbm.at[pl.ds(off, CH)], idx_v)
        pltpu.sync_copy(rows_v, acc_sh.at[idx_v], add=True)
        return carry

    lax.fori_loop(0, nk, step, 0)
    plsc.subcore_barrier()
    pltpu.sync_copy(acc_sh.at[pl.ds(s * RPT, RPT)],
                    out_hbm.at[c, pl.ds(s * RPT, RPT)])


@functools.lru_cache(maxsize=None)
def _build_scatter():
    return pl.kernel(
        _scatter_body,
        out_type=[jax.ShapeDtypeStruct((NC, NPAD, D), jnp.float32)],
        mesh=plsc.VectorSubcoreMesh(core_axis_name="c", subcore_axis_name="s"),
        scratch_types=[pltpu.VMEM((CH,), jnp.int32),
                       pltpu.VMEM((CH, D), jnp.float32),
                       pltpu.VMEM_SHARED((NPAD, D), jnp.float32)],
    )


def _scatter(msg, dst, zeros_pad):
    return _build_scatter()(msg, dst, zeros_pad)


BE = 1280


def _mlp_body(hd_ref, hs_ref, ea_ref, ax_ref, W1_ref, b1_ref, W2_ref, b2_ref,
              out_ref):
    x = jnp.concatenate(
        [hd_ref[...].astype(jnp.bfloat16), hs_ref[...].astype(jnp.bfloat16),
         ea_ref[...], ax_ref[...]], axis=1)
    acc = jnp.dot(x, W1_ref[...], preferred_element_type=jnp.float32)
    acc += b1_ref[...]
    hdn = jnp.maximum(acc, 0.0).astype(jnp.bfloat16)
    out_ref[...] = jnp.dot(hdn, W2_ref[...],
                           preferred_element_type=jnp.float32) + b2_ref[...]


def _edge_mlp(hd, hs, ea, ax, W1r, b1r, W2r, b2r):
    n_in = 2 * D + ea.shape[1] + ax.shape[1]
    hid = W1r.shape[1]
    grid = N_EDGES // BE
    return pl.pallas_call(
        _mlp_body,
        grid=(grid,),
        in_specs=[
            pl.BlockSpec((BE, D), lambda i: (i, 0)),
            pl.BlockSpec((BE, D), lambda i: (i, 0)),
            pl.BlockSpec((BE, ea.shape[1]), lambda i: (i, 0)),
            pl.BlockSpec((BE, ax.shape[1]), lambda i: (i, 0)),
            pl.BlockSpec((n_in, hid), lambda i: (0, 0)),
            pl.BlockSpec((1, hid), lambda i: (0, 0)),
            pl.BlockSpec((hid, D), lambda i: (0, 0)),
            pl.BlockSpec((1, D), lambda i: (0, 0)),
        ],
        out_specs=pl.BlockSpec((BE, D), lambda i: (i, 0)),
        out_shape=jax.ShapeDtypeStruct((N_EDGES, D), jnp.float32),
    )(hd, hs, ea, ax, W1r, b1r, W2r, b2r)


BN = 2000


def _gru_body(ap_ref, h_ref, Wih_ref, bih_ref, Whh_ref, bhh_ref, out_ref):
    a = ap_ref[0] + ap_ref[1]
    h = h_ref[...]
    gi = lax.dot_general(a, Wih_ref[...], (((1,), (1,)), ((), ())),
                         preferred_element_type=jnp.float32) + bih_ref[...]
    gh = lax.dot_general(h, Whh_ref[...], (((1,), (1,)), ((), ())),
                         preferred_element_type=jnp.float32) + bhh_ref[...]
    r = jax.nn.sigmoid(gi[:, 0:D] + gh[:, 0:D])
    z = jax.nn.sigmoid(gi[:, D:2 * D] + gh[:, D:2 * D])
    n = jnp.tanh(gi[:, 2 * D:3 * D] + r * gh[:, 2 * D:3 * D])
    out_ref[...] = (1.0 - z) * n + z * h


def _gru(ap, h, Wihr, bihr, Whhr, bhhr):
    grid = N_NODES // BN
    return pl.pallas_call(
        _gru_body,
        grid=(grid,),
        in_specs=[
            pl.BlockSpec((NC, BN, D), lambda i: (0, i, 0)),
            pl.BlockSpec((BN, D), lambda i: (i, 0)),
            pl.BlockSpec((3 * D, D), lambda i: (0, 0)),
            pl.BlockSpec((1, 3 * D), lambda i: (0, 0)),
            pl.BlockSpec((3 * D, D), lambda i: (0, 0)),
            pl.BlockSpec((1, 3 * D), lambda i: (0, 0)),
        ],
        out_specs=pl.BlockSpec((BN, D), lambda i: (i, 0)),
        out_shape=jax.ShapeDtypeStruct((N_NODES, D), jnp.float32),
    )(ap, h, Wihr, bihr, Whhr, bhhr)


def kernel(x, edge_index, edge_attr, auxiliary, W1, b1, W2, b2,
           W_ih, b_ih, W_hh, b_hh):
    ei = edge_index.astype(jnp.int32)
    src = ei[0]
    dst = ei[1]
    zeros_pad = jnp.zeros((NPAD, D), jnp.float32)
    ea_bf = edge_attr.astype(jnp.bfloat16)
    ax_bf = auxiliary.astype(jnp.bfloat16)
    W1_bf = W1.astype(jnp.bfloat16)
    W2_bf = W2.astype(jnp.bfloat16)
    h = x
    for r in range(W1.shape[0]):
        hd, hs = _gather(h, dst, src)
        msg = _edge_mlp(hd, hs, ea_bf, ax_bf,
                        W1_bf[r], b1[r].reshape(1, -1),
                        W2_bf[r], b2[r].reshape(1, -1))
        (ap,) = _scatter(msg, dst, zeros_pad)
        h = _gru(ap, h, W_ih[r], b_ih[r].reshape(1, -1),
                 W_hh[r], b_hh[r].reshape(1, -1))
    return h


# speedup vs baseline: 3.1610x; 1.1918x over previous
import functools

import jax
import jax.numpy as jnp
from jax import lax
from jax.experimental import pallas as pl
from jax.experimental.pallas import tpu as pltpu
from jax.experimental.pallas import tpu_sc as plsc

N_NODES = 10000
N_EDGES = 160000
D = 128
NPAD = 10240

NC, NS = 2, 16
NW = NC * NS
CH = 128
RPT = NPAD // NS

KSLICE = 2
NE_S = N_EDGES // KSLICE


def _worker_id():
    return lax.axis_index("s") * NC + lax.axis_index("c")


def _num_chunks(wid, nchunk):
    base = nchunk // NW
    rem = nchunk - base * NW
    return base + (wid < rem).astype(jnp.int32)


def _gather_body(h_hbm, dst_hbm, src_hbm, hd_hbm, hs_hbm,
                 idx_d, rows_d, idx_s, rows_s, sem_d, sem_s):
    wid = _worker_id()
    nk = _num_chunks(wid, dst_hbm.shape[0] // CH)

    def step(j, carry):
        off = (j * NW + wid) * CH
        pltpu.sync_copy(dst_hbm.at[pl.ds(off, CH)], idx_d)
        cp_d = pltpu.async_copy(h_hbm.at[idx_d], rows_d, sem_d)
        pltpu.sync_copy(src_hbm.at[pl.ds(off, CH)], idx_s)
        cp_s = pltpu.async_copy(h_hbm.at[idx_s], rows_s, sem_s)
        cp_d.wait()
        pltpu.sync_copy(rows_d, hd_hbm.at[pl.ds(off, CH)])
        cp_s.wait()
        pltpu.sync_copy(rows_s, hs_hbm.at[pl.ds(off, CH)])
        return carry

    lax.fori_loop(0, nk, step, 0)


@functools.lru_cache(maxsize=None)
def _build_gather(ne):
    return pl.kernel(
        _gather_body,
        out_type=[jax.ShapeDtypeStruct((ne, D), jnp.float32),
                  jax.ShapeDtypeStruct((ne, D), jnp.float32)],
        mesh=plsc.VectorSubcoreMesh(core_axis_name="c", subcore_axis_name="s"),
        scratch_types=[pltpu.VMEM((CH,), jnp.int32),
                       pltpu.VMEM((CH, D), jnp.float32),
                       pltpu.VMEM((CH,), jnp.int32),
                       pltpu.VMEM((CH, D), jnp.float32),
                       pltpu.SemaphoreType.DMA,
                       pltpu.SemaphoreType.DMA],
    )


def _gather(h, dst, src):
    return _build_gather(dst.shape[0])(h, dst, src)


def _scatter_body(msg_hbm, dst_hbm, zeros_hbm, out_hbm, idx_v, rows_v, acc_sh):
    c = lax.axis_index("c")
    s = lax.axis_index("s")
    wid = _worker_id()
    nk = _num_chunks(wid, dst_hbm.shape[0] // CH)

    pltpu.sync_copy(zeros_hbm.at[pl.ds(s * RPT, RPT)],
                    acc_sh.at[pl.ds(s * RPT, RPT)])
    plsc.subcore_barrier()

    def step(j, carry):
        off = (j * NW + wid) * CH
        pltpu.sync_copy(msg_hbm.at[pl.ds(off, CH)], rows_v)
        pltpu.sync_copy(dst_hbm.at[pl.ds(off, CH)], idx_v)
        pltpu.sync_copy(rows_v, acc_sh.at[idx_v], add=True)
        return carry

    lax.fori_loop(0, nk, step, 0)
    plsc.subcore_barrier()
    pltpu.sync_copy(acc_sh.at[pl.ds(s * RPT, RPT)],
                    out_hbm.at[c, pl.ds(s * RPT, RPT)])


@functools.lru_cache(maxsize=None)
def _build_scatter(ne):
    return pl.kernel(
        _scatter_body,
        out_type=[jax.ShapeDtypeStruct((NC, NPAD, D), jnp.float32)],
        mesh=plsc.VectorSubcoreMesh(core_axis_name="c", subcore_axis_name="s"),
        scratch_types=[pltpu.VMEM((CH,), jnp.int32),
                       pltpu.VMEM((CH, D), jnp.float32),
                       pltpu.VMEM_SHARED((NPAD, D), jnp.float32)],
    )


def _scatter(msg, dst, zeros_pad):
    return _build_scatter(dst.shape[0])(msg, dst, zeros_pad)


BE = 1280


def _mlp_body(hd_ref, hs_ref, ea_ref, ax_ref, W1_ref, b1_ref, W2_ref, b2_ref,
              out_ref):
    x = jnp.concatenate(
        [hd_ref[...].astype(jnp.bfloat16), hs_ref[...].astype(jnp.bfloat16),
         ea_ref[...], ax_ref[...]], axis=1)
    acc = jnp.dot(x, W1_ref[...], preferred_element_type=jnp.float32)
    acc += b1_ref[...]
    hdn = jnp.maximum(acc, 0.0).astype(jnp.bfloat16)
    out_ref[...] = jnp.dot(hdn, W2_ref[...],
                           preferred_element_type=jnp.float32) + b2_ref[...]


def _edge_mlp(hd, hs, ea, ax, W1r, b1r, W2r, b2r, blk_off):
    n_in = 2 * D + ea.shape[1] + ax.shape[1]
    hid = W1r.shape[1]
    ne = hd.shape[0]
    grid = ne // BE
    shifted = lambda i, o=blk_off: (i + o, 0)
    local = lambda i: (i, 0)
    return pl.pallas_call(
        _mlp_body,
        grid=(grid,),
        in_specs=[
            pl.BlockSpec((BE, D), local),
            pl.BlockSpec((BE, D), local),
            pl.BlockSpec((BE, ea.shape[1]), shifted),
            pl.BlockSpec((BE, ax.shape[1]), shifted),
            pl.BlockSpec((n_in, hid), lambda i: (0, 0)),
            pl.BlockSpec((1, hid), lambda i: (0, 0)),
            pl.BlockSpec((hid, D), lambda i: (0, 0)),
            pl.BlockSpec((1, D), lambda i: (0, 0)),
        ],
        out_specs=pl.BlockSpec((BE, D), local),
        out_shape=jax.ShapeDtypeStruct((ne, D), jnp.float32),
    )(hd, hs, ea, ax, W1r, b1r, W2r, b2r)


BN = 2000


def _gru_body(ap0_ref, ap1_ref, h_ref, Wih_ref, bih_ref, Whh_ref, bhh_ref,
              out_ref):
    a = ap0_ref[0] + ap0_ref[1] + ap1_ref[0] + ap1_ref[1]
    h = h_ref[...]
    gi = lax.dot_general(a, Wih_ref[...], (((1,), (1,)), ((), ())),
                         preferred_element_type=jnp.float32) + bih_ref[...]
    gh = lax.dot_general(h, Whh_ref[...], (((1,), (1,)), ((), ())),
                         preferred_element_type=jnp.float32) + bhh_ref[...]
    r = jax.nn.sigmoid(gi[:, 0:D] + gh[:, 0:D])
    z = jax.nn.sigmoid(gi[:, D:2 * D] + gh[:, D:2 * D])
    n = jnp.tanh(gi[:, 2 * D:3 * D] + r * gh[:, 2 * D:3 * D])
    out_ref[...] = (1.0 - z) * n + z * h


def _gru(ap0, ap1, h, Wihr, bihr, Whhr, bhhr):
    grid = N_NODES // BN
    ap_spec = pl.BlockSpec((NC, BN, D), lambda i: (0, i, 0))
    return pl.pallas_call(
        _gru_body,
        grid=(grid,),
        in_specs=[
            ap_spec,
            ap_spec,
            pl.BlockSpec((BN, D), lambda i: (i, 0)),
            pl.BlockSpec((3 * D, D), lambda i: (0, 0)),
            pl.BlockSpec((1, 3 * D), lambda i: (0, 0)),
            pl.BlockSpec((3 * D, D), lambda i: (0, 0)),
            pl.BlockSpec((1, 3 * D), lambda i: (0, 0)),
        ],
        out_specs=pl.BlockSpec((BN, D), lambda i: (i, 0)),
        out_shape=jax.ShapeDtypeStruct((N_NODES, D), jnp.float32),
    )(ap0, ap1, h, Wihr, bihr, Whhr, bhhr)


def kernel(x, edge_index, edge_attr, auxiliary, W1, b1, W2, b2,
           W_ih, b_ih, W_hh, b_hh):
    ei = edge_index.astype(jnp.int32)
    src = ei[0]
    dst = ei[1]
    dst_s = [lax.slice(dst, (k * NE_S,), ((k + 1) * NE_S,))
             for k in range(KSLICE)]
    src_s = [lax.slice(src, (k * NE_S,), ((k + 1) * NE_S,))
             for k in range(KSLICE)]
    zeros_pad = jnp.zeros((NPAD, D), jnp.float32)
    ea_bf = edge_attr.astype(jnp.bfloat16)
    ax_bf = auxiliary.astype(jnp.bfloat16)
    W1_bf = W1.astype(jnp.bfloat16)
    W2_bf = W2.astype(jnp.bfloat16)
    h = x
    for r in range(W1.shape[0]):
        aps = []
        for k in range(KSLICE):
            hd, hs = _gather(h, dst_s[k], src_s[k])
            msg = _edge_mlp(hd, hs, ea_bf, ax_bf,
                            W1_bf[r], b1[r].reshape(1, -1),
                            W2_bf[r], b2[r].reshape(1, -1),
                            k * (NE_S // BE))
            (ap,) = _scatter(msg, dst_s[k], zeros_pad)
            aps.append(ap)
        h = _gru(aps[0], aps[1], h, W_ih[r], b_ih[r].reshape(1, -1),
                 W_hh[r], b_hh[r].reshape(1, -1))
    return h


# speedup vs baseline: 3.2373x; 1.0242x over previous
import functools

import jax
import jax.numpy as jnp
from jax import lax
from jax.experimental import pallas as pl
from jax.experimental.pallas import tpu as pltpu
from jax.experimental.pallas import tpu_sc as plsc

N_NODES = 10000
N_EDGES = 160000
D = 128
NPAD = 10240

NC, NS = 2, 16
NW = NC * NS
CH = 128
RPT = NPAD // NS

KSLICE = 2
NE_S = N_EDGES // KSLICE


def _worker_id():
    return lax.axis_index("s") * NC + lax.axis_index("c")


def _num_chunks(wid, nchunk):
    base = nchunk // NW
    rem = nchunk - base * NW
    return base + (wid < rem).astype(jnp.int32)


def _gather_body(h_hbm, dst_hbm, src_hbm, hd_hbm, hs_hbm,
                 idx_d, rows_d, idx_s, rows_s, sem_d, sem_s):
    wid = _worker_id()
    nk = _num_chunks(wid, dst_hbm.shape[0] // CH)

    def step(j, carry):
        off = (j * NW + wid) * CH
        pltpu.sync_copy(dst_hbm.at[pl.ds(off, CH)], idx_d)
        cp_d = pltpu.async_copy(h_hbm.at[idx_d], rows_d, sem_d)
        pltpu.sync_copy(src_hbm.at[pl.ds(off, CH)], idx_s)
        cp_s = pltpu.async_copy(h_hbm.at[idx_s], rows_s, sem_s)
        cp_d.wait()
        pltpu.sync_copy(rows_d, hd_hbm.at[pl.ds(off, CH)])
        cp_s.wait()
        pltpu.sync_copy(rows_s, hs_hbm.at[pl.ds(off, CH)])
        return carry

    lax.fori_loop(0, nk, step, 0)


@functools.lru_cache(maxsize=None)
def _build_gather(ne):
    return pl.kernel(
        _gather_body,
        out_type=[jax.ShapeDtypeStruct((ne, D), jnp.float32),
                  jax.ShapeDtypeStruct((ne, D), jnp.float32)],
        mesh=plsc.VectorSubcoreMesh(core_axis_name="c", subcore_axis_name="s"),
        scratch_types=[pltpu.VMEM((CH,), jnp.int32),
                       pltpu.VMEM((CH, D), jnp.float32),
                       pltpu.VMEM((CH,), jnp.int32),
                       pltpu.VMEM((CH, D), jnp.float32),
                       pltpu.SemaphoreType.DMA,
                       pltpu.SemaphoreType.DMA],
    )


def _gather(h, dst, src):
    return _build_gather(dst.shape[0])(h, dst, src)


def _scatter_body(msg_hbm, dst_hbm, zeros_hbm, out_hbm, idx_v, rows_v, acc_sh):
    c = lax.axis_index("c")
    s = lax.axis_index("s")
    wid = _worker_id()
    nk = _num_chunks(wid, dst_hbm.shape[0] // CH)

    pltpu.sync_copy(zeros_hbm.at[pl.ds(s * RPT, RPT)],
                    acc_sh.at[pl.ds(s * RPT, RPT)])
    plsc.subcore_barrier()

    def step(j, carry):
        off = (j * NW + wid) * CH
        pltpu.sync_copy(msg_hbm.at[pl.ds(off, CH)], rows_v)
        pltpu.sync_copy(dst_hbm.at[pl.ds(off, CH)], idx_v)
        pltpu.sync_copy(rows_v, acc_sh.at[idx_v], add=True)
        return carry

    lax.fori_loop(0, nk, step, 0)
    plsc.subcore_barrier()
    pltpu.sync_copy(acc_sh.at[pl.ds(s * RPT, RPT)],
                    out_hbm.at[c, pl.ds(s * RPT, RPT)])


@functools.lru_cache(maxsize=None)
def _build_scatter(ne):
    return pl.kernel(
        _scatter_body,
        out_type=[jax.ShapeDtypeStruct((NC, NPAD, D), jnp.float32)],
        mesh=plsc.VectorSubcoreMesh(core_axis_name="c", subcore_axis_name="s"),
        scratch_types=[pltpu.VMEM((CH,), jnp.int32),
                       pltpu.VMEM((CH, D), jnp.float32),
                       pltpu.VMEM_SHARED((NPAD, D), jnp.float32)],
    )


def _scatter(msg, dst, zeros_pad):
    return _build_scatter(dst.shape[0])(msg, dst, zeros_pad)


BE = 1600


def _mlp_body(hd_ref, hs_ref, ea_ref, ax_ref, W1_ref, b1_ref, W2_ref, b2_ref,
              out_ref):
    x = jnp.concatenate(
        [hd_ref[...].astype(jnp.bfloat16), hs_ref[...].astype(jnp.bfloat16),
         ea_ref[...], ax_ref[...]], axis=1)
    acc = jnp.dot(x, W1_ref[...], preferred_element_type=jnp.float32)
    acc += b1_ref[...]
    hdn = jnp.maximum(acc, 0.0).astype(jnp.bfloat16)
    out_ref[...] = jnp.dot(hdn, W2_ref[...],
                           preferred_element_type=jnp.float32) + b2_ref[...]


def _edge_mlp(hd, hs, ea, ax, W1r, b1r, W2r, b2r, blk_off):
    n_in = 2 * D + ea.shape[1] + ax.shape[1]
    hid = W1r.shape[1]
    ne = hd.shape[0]
    grid = ne // BE
    shifted = lambda i, o=blk_off: (i + o, 0)
    local = lambda i: (i, 0)
    return pl.pallas_call(
        _mlp_body,
        grid=(grid,),
        in_specs=[
            pl.BlockSpec((BE, D), local),
            pl.BlockSpec((BE, D), local),
            pl.BlockSpec((BE, ea.shape[1]), shifted),
            pl.BlockSpec((BE, ax.shape[1]), shifted),
            pl.BlockSpec((n_in, hid), lambda i: (0, 0)),
            pl.BlockSpec((1, hid), lambda i: (0, 0)),
            pl.BlockSpec((hid, D), lambda i: (0, 0)),
            pl.BlockSpec((1, D), lambda i: (0, 0)),
        ],
        out_specs=pl.BlockSpec((BE, D), local),
        out_shape=jax.ShapeDtypeStruct((ne, D), jnp.float32),
    )(hd, hs, ea, ax, W1r, b1r, W2r, b2r)


BN = 2000


def _gru_body(ap0_ref, ap1_ref, h_ref, Wih_ref, bih_ref, Whh_ref, bhh_ref,
              out_ref):
    a = ap0_ref[0] + ap0_ref[1] + ap1_ref[0] + ap1_ref[1]
    h = h_ref[...]
    gi = lax.dot_general(a, Wih_ref[...], (((1,), (1,)), ((), ())),
                         preferred_element_type=jnp.float32) + bih_ref[...]
    gh = lax.dot_general(h, Whh_ref[...], (((1,), (1,)), ((), ())),
                         preferred_element_type=jnp.float32) + bhh_ref[...]
    r = jax.nn.sigmoid(gi[:, 0:D] + gh[:, 0:D])
    z = jax.nn.sigmoid(gi[:, D:2 * D] + gh[:, D:2 * D])
    n = jnp.tanh(gi[:, 2 * D:3 * D] + r * gh[:, 2 * D:3 * D])
    out_ref[...] = (1.0 - z) * n + z * h


def _gru(ap0, ap1, h, Wihr, bihr, Whhr, bhhr):
    grid = N_NODES // BN
    ap_spec = pl.BlockSpec((NC, BN, D), lambda i: (0, i, 0))
    return pl.pallas_call(
        _gru_body,
        grid=(grid,),
        in_specs=[
            ap_spec,
            ap_spec,
            pl.BlockSpec((BN, D), lambda i: (i, 0)),
            pl.BlockSpec((3 * D, D), lambda i: (0, 0)),
            pl.BlockSpec((1, 3 * D), lambda i: (0, 0)),
            pl.BlockSpec((3 * D, D), lambda i: (0, 0)),
            pl.BlockSpec((1, 3 * D), lambda i: (0, 0)),
        ],
        out_specs=pl.BlockSpec((BN, D), lambda i: (i, 0)),
        out_shape=jax.ShapeDtypeStruct((N_NODES, D), jnp.float32),
    )(ap0, ap1, h, Wihr, bihr, Whhr, bhhr)


def kernel(x, edge_index, edge_attr, auxiliary, W1, b1, W2, b2,
           W_ih, b_ih, W_hh, b_hh):
    ei = edge_index.astype(jnp.int32)
    src = ei[0]
    dst = ei[1]
    dst_s = [lax.slice(dst, (k * NE_S,), ((k + 1) * NE_S,))
             for k in range(KSLICE)]
    src_s = [lax.slice(src, (k * NE_S,), ((k + 1) * NE_S,))
             for k in range(KSLICE)]
    zeros_pad = jnp.zeros((NPAD, D), jnp.float32)
    ea_bf = edge_attr.astype(jnp.bfloat16)
    ax_bf = auxiliary.astype(jnp.bfloat16)
    W1_bf = W1.astype(jnp.bfloat16)
    W2_bf = W2.astype(jnp.bfloat16)
    h = x
    for r in range(W1.shape[0]):
        aps = []
        for k in range(KSLICE):
            hd, hs = _gather(h, dst_s[k], src_s[k])
            msg = _edge_mlp(hd, hs, ea_bf, ax_bf,
                            W1_bf[r], b1[r].reshape(1, -1),
                            W2_bf[r], b2[r].reshape(1, -1),
                            k * (NE_S // BE))
            (ap,) = _scatter(msg, dst_s[k], zeros_pad)
            aps.append(ap)
        h = _gru(aps[0], aps[1], h, W_ih[r], b_ih[r].reshape(1, -1),
                 W_hh[r], b_hh[r].reshape(1, -1))
    return h


# speedup vs baseline: 3.2610x; 1.0073x over previous
import functools

import jax
import jax.numpy as jnp
from jax import lax
from jax.experimental import pallas as pl
from jax.experimental.pallas import tpu as pltpu
from jax.experimental.pallas import tpu_sc as plsc

N_NODES = 10000
N_EDGES = 160000
D = 128
NPAD = 10240

NC, NS = 2, 16
NW = NC * NS
CH = 128
RPT = NPAD // NS

KSLICE = 2
NE_S = N_EDGES // KSLICE


def _worker_id():
    return lax.axis_index("s") * NC + lax.axis_index("c")


def _num_chunks(wid, nchunk):
    base = nchunk // NW
    rem = nchunk - base * NW
    return base + (wid < rem).astype(jnp.int32)


def _gather_body(h_hbm, dst_hbm, src_hbm, hd_hbm, hs_hbm,
                 idx_d, rows_d, idx_s, rows_s, sem_d, sem_s):
    wid = _worker_id()
    nk = _num_chunks(wid, dst_hbm.shape[0] // CH)

    def step(j, carry):
        off = (j * NW + wid) * CH
        pltpu.sync_copy(dst_hbm.at[pl.ds(off, CH)], idx_d)
        cp_d = pltpu.async_copy(h_hbm.at[idx_d], rows_d, sem_d)
        pltpu.sync_copy(src_hbm.at[pl.ds(off, CH)], idx_s)
        cp_s = pltpu.async_copy(h_hbm.at[idx_s], rows_s, sem_s)
        cp_d.wait()
        pltpu.sync_copy(rows_d, hd_hbm.at[pl.ds(off, CH)])
        cp_s.wait()
        pltpu.sync_copy(rows_s, hs_hbm.at[pl.ds(off, CH)])
        return carry

    lax.fori_loop(0, nk, step, 0)


@functools.lru_cache(maxsize=None)
def _build_gather(ne):
    return pl.kernel(
        _gather_body,
        out_type=[jax.ShapeDtypeStruct((ne, D), jnp.float32),
                  jax.ShapeDtypeStruct((ne, D), jnp.float32)],
        mesh=plsc.VectorSubcoreMesh(core_axis_name="c", subcore_axis_name="s"),
        scratch_types=[pltpu.VMEM((CH,), jnp.int32),
                       pltpu.VMEM((CH, D), jnp.float32),
                       pltpu.VMEM((CH,), jnp.int32),
                       pltpu.VMEM((CH, D), jnp.float32),
                       pltpu.SemaphoreType.DMA,
                       pltpu.SemaphoreType.DMA],
    )


def _gather(h, dst, src):
    return _build_gather(dst.shape[0])(h, dst, src)


def _scatter_body(msg_hbm, dst_hbm, zeros_hbm, out_hbm, idx_v, rows_v, acc_sh):
    c = lax.axis_index("c")
    s = lax.axis_index("s")
    wid = _worker_id()
    nk = _num_chunks(wid, dst_hbm.shape[0] // CH)

    pltpu.sync_copy(zeros_hbm.at[pl.ds(s * RPT, RPT)],
                    acc_sh.at[pl.ds(s * RPT, RPT)])
    plsc.subcore_barrier()

    def step(j, carry):
        off = (j * NW + wid) * CH
        pltpu.sync_copy(msg_hbm.at[pl.ds(off, CH)], rows_v)
        pltpu.sync_copy(dst_hbm.at[pl.ds(off, CH)], idx_v)
        pltpu.sync_copy(rows_v, acc_sh.at[idx_v], add=True)
        return carry

    lax.fori_loop(0, nk, step, 0)
    plsc.subcore_barrier()
    pltpu.sync_copy(acc_sh.at[pl.ds(s * RPT, RPT)],
                    out_hbm.at[c, pl.ds(s * RPT, RPT)])


@functools.lru_cache(maxsize=None)
def _build_scatter(ne):
    return pl.kernel(
        _scatter_body,
        out_type=[jax.ShapeDtypeStruct((NC, NPAD, D), jnp.float32)],
        mesh=plsc.VectorSubcoreMesh(core_axis_name="c", subcore_axis_name="s"),
        scratch_types=[pltpu.VMEM((CH,), jnp.int32),
                       pltpu.VMEM((CH, D), jnp.float32),
                       pltpu.VMEM_SHARED((NPAD, D), jnp.float32)],
    )


def _scatter(msg, dst, zeros_pad):
    return _build_scatter(dst.shape[0])(msg, dst, zeros_pad)


BE = 1600


def _mlp_body(hd_ref, hs_ref, ea_ref, ax_ref, W1_ref, b1_ref, W2_ref, b2_ref,
              out_ref):
    x = jnp.concatenate(
        [hd_ref[...].astype(jnp.bfloat16), hs_ref[...].astype(jnp.bfloat16),
         ea_ref[...].astype(jnp.bfloat16), ax_ref[...].astype(jnp.bfloat16)],
        axis=1)
    acc = jnp.dot(x, W1_ref[...], preferred_element_type=jnp.float32)
    acc += b1_ref[...]
    hdn = jnp.maximum(acc, 0.0).astype(jnp.bfloat16)
    out_ref[...] = jnp.dot(hdn, W2_ref[...],
                           preferred_element_type=jnp.float32) + b2_ref[...]


def _edge_mlp(hd, hs, ea, ax, W1r, b1r, W2r, b2r, blk_off):
    n_in = 2 * D + ea.shape[1] + ax.shape[1]
    hid = W1r.shape[1]
    ne = hd.shape[0]
    grid = ne // BE
    shifted = lambda i, o=blk_off: (i + o, 0)
    local = lambda i: (i, 0)
    return pl.pallas_call(
        _mlp_body,
        grid=(grid,),
        in_specs=[
            pl.BlockSpec((BE, D), local),
            pl.BlockSpec((BE, D), local),
            pl.BlockSpec((BE, ea.shape[1]), shifted),
            pl.BlockSpec((BE, ax.shape[1]), shifted),
            pl.BlockSpec((n_in, hid), lambda i: (0, 0)),
            pl.BlockSpec((1, hid), lambda i: (0, 0)),
            pl.BlockSpec((hid, D), lambda i: (0, 0)),
            pl.BlockSpec((1, D), lambda i: (0, 0)),
        ],
        out_specs=pl.BlockSpec((BE, D), local),
        out_shape=jax.ShapeDtypeStruct((ne, D), jnp.float32),
    )(hd, hs, ea, ax, W1r, b1r, W2r, b2r)


BN = 2000


def _gru_body(ap0_ref, ap1_ref, h_ref, Wih_ref, bih_ref, Whh_ref, bhh_ref,
              out_ref):
    a = ap0_ref[0] + ap0_ref[1] + ap1_ref[0] + ap1_ref[1]
    h = h_ref[...]
    gi = lax.dot_general(a, Wih_ref[...], (((1,), (1,)), ((), ())),
                         preferred_element_type=jnp.float32) + bih_ref[...]
    gh = lax.dot_general(h, Whh_ref[...], (((1,), (1,)), ((), ())),
                         preferred_element_type=jnp.float32) + bhh_ref[...]
    r = jax.nn.sigmoid(gi[:, 0:D] + gh[:, 0:D])
    z = jax.nn.sigmoid(gi[:, D:2 * D] + gh[:, D:2 * D])
    n = jnp.tanh(gi[:, 2 * D:3 * D] + r * gh[:, 2 * D:3 * D])
    out_ref[...] = (1.0 - z) * n + z * h


def _gru(ap0, ap1, h, Wihr, bihr, Whhr, bhhr):
    grid = N_NODES // BN
    ap_spec = pl.BlockSpec((NC, BN, D), lambda i: (0, i, 0))
    return pl.pallas_call(
        _gru_body,
        grid=(grid,),
        in_specs=[
            ap_spec,
            ap_spec,
            pl.BlockSpec((BN, D), lambda i: (i, 0)),
            pl.BlockSpec((3 * D, D), lambda i: (0, 0)),
            pl.BlockSpec((1, 3 * D), lambda i: (0, 0)),
            pl.BlockSpec((3 * D, D), lambda i: (0, 0)),
            pl.BlockSpec((1, 3 * D), lambda i: (0, 0)),
        ],
        out_specs=pl.BlockSpec((BN, D), lambda i: (i, 0)),
        out_shape=jax.ShapeDtypeStruct((N_NODES, D), jnp.float32),
    )(ap0, ap1, h, Wihr, bihr, Whhr, bhhr)


def kernel(x, edge_index, edge_attr, auxiliary, W1, b1, W2, b2,
           W_ih, b_ih, W_hh, b_hh):
    ei = edge_index.astype(jnp.int32)
    src = ei[0]
    dst = ei[1]
    dst_s = [lax.slice(dst, (k * NE_S,), ((k + 1) * NE_S,))
             for k in range(KSLICE)]
    src_s = [lax.slice(src, (k * NE_S,), ((k + 1) * NE_S,))
             for k in range(KSLICE)]
    zeros_pad = jnp.zeros((NPAD, D), jnp.float32)
    W1_bf = W1.astype(jnp.bfloat16)
    W2_bf = W2.astype(jnp.bfloat16)
    h = x
    for r in range(W1.shape[0]):
        aps = []
        for k in range(KSLICE):
            hd, hs = _gather(h, dst_s[k], src_s[k])
            msg = _edge_mlp(hd, hs, edge_attr, auxiliary,
                            W1_bf[r], b1[r].reshape(1, -1),
                            W2_bf[r], b2[r].reshape(1, -1),
                            k * (NE_S // BE))
            (ap,) = _scatter(msg, dst_s[k], zeros_pad)
            aps.append(ap)
        h = _gru(aps[0], aps[1], h, W_ih[r], b_ih[r].reshape(1, -1),
                 W_hh[r], b_hh[r].reshape(1, -1))
    return h


# speedup vs baseline: 3.4141x; 1.0470x over previous
import functools

import jax
import jax.numpy as jnp
from jax import lax
from jax.experimental import pallas as pl
from jax.experimental.pallas import tpu as pltpu
from jax.experimental.pallas import tpu_sc as plsc

N_NODES = 10000
N_EDGES = 160000
D = 128
NPAD = 10240

NC, NS = 2, 16
NW = NC * NS
CH = 128
RPT = NPAD // NS

KSLICE = 2
NE_S = N_EDGES // KSLICE


def _worker_id():
    return lax.axis_index("s") * NC + lax.axis_index("c")


def _num_chunks(wid, nchunk):
    base = nchunk // NW
    rem = nchunk - base * NW
    return base + (wid < rem).astype(jnp.int32)


def _gather_body(h_hbm, dst_hbm, src_hbm, hd_hbm, hs_hbm,
                 idx_d, rows_d, idx_s, rows_s, sem_d, sem_s):
    wid = _worker_id()
    nk = _num_chunks(wid, dst_hbm.shape[0] // CH)

    def step(j, carry):
        off = (j * NW + wid) * CH
        pltpu.sync_copy(dst_hbm.at[pl.ds(off, CH)], idx_d)
        cp_d = pltpu.async_copy(h_hbm.at[idx_d], rows_d, sem_d)
        pltpu.sync_copy(src_hbm.at[pl.ds(off, CH)], idx_s)
        cp_s = pltpu.async_copy(h_hbm.at[idx_s], rows_s, sem_s)
        cp_d.wait()
        pltpu.sync_copy(rows_d, hd_hbm.at[pl.ds(off, CH)])
        cp_s.wait()
        pltpu.sync_copy(rows_s, hs_hbm.at[pl.ds(off, CH)])
        return carry

    lax.fori_loop(0, nk, step, 0)


@functools.lru_cache(maxsize=None)
def _build_gather(ne):
    return pl.kernel(
        _gather_body,
        out_type=[jax.ShapeDtypeStruct((ne, D), jnp.float32),
                  jax.ShapeDtypeStruct((ne, D), jnp.float32)],
        mesh=plsc.VectorSubcoreMesh(core_axis_name="c", subcore_axis_name="s"),
        scratch_types=[pltpu.VMEM((CH,), jnp.int32),
                       pltpu.VMEM((CH, D), jnp.float32),
                       pltpu.VMEM((CH,), jnp.int32),
                       pltpu.VMEM((CH, D), jnp.float32),
                       pltpu.SemaphoreType.DMA,
                       pltpu.SemaphoreType.DMA],
    )


def _gather(h, dst, src):
    return _build_gather(dst.shape[0])(h, dst, src)


def _scatter_body(msg_hbm, dst_hbm, zeros_hbm, out_hbm, idx_v, rows_v, acc_sh):
    c = lax.axis_index("c")
    s = lax.axis_index("s")
    wid = _worker_id()
    nk = _num_chunks(wid, dst_hbm.shape[0] // CH)

    pltpu.sync_copy(zeros_hbm.at[pl.ds(s * RPT, RPT)],
                    acc_sh.at[pl.ds(s * RPT, RPT)])
    plsc.subcore_barrier()

    def step(j, carry):
        off = (j * NW + wid) * CH
        pltpu.sync_copy(msg_hbm.at[pl.ds(off, CH)], rows_v)
        pltpu.sync_copy(dst_hbm.at[pl.ds(off, CH)], idx_v)
        pltpu.sync_copy(rows_v, acc_sh.at[idx_v], add=True)
        return carry

    lax.fori_loop(0, nk, step, 0)
    plsc.subcore_barrier()
    pltpu.sync_copy(acc_sh.at[pl.ds(s * RPT, RPT)],
                    out_hbm.at[c, pl.ds(s * RPT, RPT)])


@functools.lru_cache(maxsize=None)
def _build_scatter(ne):
    return pl.kernel(
        _scatter_body,
        out_type=[jax.ShapeDtypeStruct((NC, NPAD, D), jnp.float32)],
        mesh=plsc.VectorSubcoreMesh(core_axis_name="c", subcore_axis_name="s"),
        scratch_types=[pltpu.VMEM((CH,), jnp.int32),
                       pltpu.VMEM((CH, D), jnp.float32),
                       pltpu.VMEM_SHARED((NPAD, D), jnp.float32)],
    )


def _scatter(msg, dst, zeros_pad):
    return _build_scatter(dst.shape[0])(msg, dst, zeros_pad)


BE = 3200


def _mlp_body(hd_ref, hs_ref, ea_ref, ax_ref, W1_ref, b1_ref, W2_ref, b2_ref,
              out_ref):
    x = jnp.concatenate(
        [hd_ref[...].astype(jnp.bfloat16), hs_ref[...].astype(jnp.bfloat16),
         ea_ref[...].astype(jnp.bfloat16), ax_ref[...].astype(jnp.bfloat16)],
        axis=1)
    acc = jnp.dot(x, W1_ref[...], preferred_element_type=jnp.float32)
    acc += b1_ref[...]
    hdn = jnp.maximum(acc, 0.0).astype(jnp.bfloat16)
    out_ref[...] = jnp.dot(hdn, W2_ref[...],
                           preferred_element_type=jnp.float32) + b2_ref[...]


def _edge_mlp(hd, hs, ea, ax, W1r, b1r, W2r, b2r, blk_off):
    n_in = 2 * D + ea.shape[1] + ax.shape[1]
    hid = W1r.shape[1]
    ne = hd.shape[0]
    grid = ne // BE
    shifted = lambda i, o=blk_off: (i + o, 0)
    local = lambda i: (i, 0)
    return pl.pallas_call(
        _mlp_body,
        grid=(grid,),
        in_specs=[
            pl.BlockSpec((BE, D), local),
            pl.BlockSpec((BE, D), local),
            pl.BlockSpec((BE, ea.shape[1]), shifted),
            pl.BlockSpec((BE, ax.shape[1]), shifted),
            pl.BlockSpec((n_in, hid), lambda i: (0, 0)),
            pl.BlockSpec((1, hid), lambda i: (0, 0)),
            pl.BlockSpec((hid, D), lambda i: (0, 0)),
            pl.BlockSpec((1, D), lambda i: (0, 0)),
        ],
        out_specs=pl.BlockSpec((BE, D), local),
        out_shape=jax.ShapeDtypeStruct((ne, D), jnp.float32),
    )(hd, hs, ea, ax, W1r, b1r, W2r, b2r)


BN = 2000


def _gru_body(ap0_ref, ap1_ref, h_ref, Wih_ref, bih_ref, Whh_ref, bhh_ref,
              out_ref):
    a = ap0_ref[0] + ap0_ref[1] + ap1_ref[0] + ap1_ref[1]
    h = h_ref[...]
    gi = lax.dot_general(a, Wih_ref[...], (((1,), (1,)), ((), ())),
                         preferred_element_type=jnp.float32) + bih_ref[...]
    gh = lax.dot_general(h, Whh_ref[...], (((1,), (1,)), ((), ())),
                         preferred_element_type=jnp.float32) + bhh_ref[...]
    r = jax.nn.sigmoid(gi[:, 0:D] + gh[:, 0:D])
    z = jax.nn.sigmoid(gi[:, D:2 * D] + gh[:, D:2 * D])
    n = jnp.tanh(gi[:, 2 * D:3 * D] + r * gh[:, 2 * D:3 * D])
    out_ref[...] = (1.0 - z) * n + z * h


def _gru(ap0, ap1, h, Wihr, bihr, Whhr, bhhr):
    grid = N_NODES // BN
    ap_spec = pl.BlockSpec((NC, BN, D), lambda i: (0, i, 0))
    return pl.pallas_call(
        _gru_body,
        grid=(grid,),
        in_specs=[
            ap_spec,
            ap_spec,
            pl.BlockSpec((BN, D), lambda i: (i, 0)),
            pl.BlockSpec((3 * D, D), lambda i: (0, 0)),
            pl.BlockSpec((1, 3 * D), lambda i: (0, 0)),
            pl.BlockSpec((3 * D, D), lambda i: (0, 0)),
            pl.BlockSpec((1, 3 * D), lambda i: (0, 0)),
        ],
        out_specs=pl.BlockSpec((BN, D), lambda i: (i, 0)),
        out_shape=jax.ShapeDtypeStruct((N_NODES, D), jnp.float32),
    )(ap0, ap1, h, Wihr, bihr, Whhr, bhhr)


def kernel(x, edge_index, edge_attr, auxiliary, W1, b1, W2, b2,
           W_ih, b_ih, W_hh, b_hh):
    ei = edge_index.astype(jnp.int32)
    src = ei[0]
    dst = ei[1]
    dst_s = [lax.slice(dst, (k * NE_S,), ((k + 1) * NE_S,))
             for k in range(KSLICE)]
    src_s = [lax.slice(src, (k * NE_S,), ((k + 1) * NE_S,))
             for k in range(KSLICE)]
    zeros_pad = jnp.zeros((NPAD, D), jnp.float32)
    W1_bf = W1.astype(jnp.bfloat16)
    W2_bf = W2.astype(jnp.bfloat16)
    h = x
    for r in range(W1.shape[0]):
        aps = []
        for k in range(KSLICE):
            hd, hs = _gather(h, dst_s[k], src_s[k])
            msg = _edge_mlp(hd, hs, edge_attr, auxiliary,
                            W1_bf[r], b1[r].reshape(1, -1),
                            W2_bf[r], b2[r].reshape(1, -1),
                            k * (NE_S // BE))
            (ap,) = _scatter(msg, dst_s[k], zeros_pad)
            aps.append(ap)
        h = _gru(aps[0], aps[1], h, W_ih[r], b_ih[r].reshape(1, -1),
                 W_hh[r], b_hh[r].reshape(1, -1))
    return h


# speedup vs baseline: 3.4179x; 1.0011x over previous
import functools

import jax
import jax.numpy as jnp
from jax import lax
from jax.experimental import pallas as pl
from jax.experimental.pallas import tpu as pltpu
from jax.experimental.pallas import tpu_sc as plsc

N_NODES = 10000
N_EDGES = 160000
D = 128
NPAD = 10240

NC, NS = 2, 16
NW = NC * NS
CH = 128
RPT = NPAD // NS

KSLICE = 5
NE_S = N_EDGES // KSLICE


def _worker_id():
    return lax.axis_index("s") * NC + lax.axis_index("c")


def _num_chunks(wid, nchunk):
    base = nchunk // NW
    rem = nchunk - base * NW
    return base + (wid < rem).astype(jnp.int32)


def _gather_body(h_hbm, dst_hbm, src_hbm, hd_hbm, hs_hbm,
                 idx_d, rows_d, idx_s, rows_s, sem_d, sem_s):
    wid = _worker_id()
    nk = _num_chunks(wid, dst_hbm.shape[0] // CH)

    def step(j, carry):
        off = (j * NW + wid) * CH
        pltpu.sync_copy(dst_hbm.at[pl.ds(off, CH)], idx_d)
        cp_d = pltpu.async_copy(h_hbm.at[idx_d], rows_d, sem_d)
        pltpu.sync_copy(src_hbm.at[pl.ds(off, CH)], idx_s)
        cp_s = pltpu.async_copy(h_hbm.at[idx_s], rows_s, sem_s)
        cp_d.wait()
        pltpu.sync_copy(rows_d, hd_hbm.at[pl.ds(off, CH)])
        cp_s.wait()
        pltpu.sync_copy(rows_s, hs_hbm.at[pl.ds(off, CH)])
        return carry

    lax.fori_loop(0, nk, step, 0)


@functools.lru_cache(maxsize=None)
def _build_gather(ne):
    return pl.kernel(
        _gather_body,
        out_type=[jax.ShapeDtypeStruct((ne, D), jnp.float32),
                  jax.ShapeDtypeStruct((ne, D), jnp.float32)],
        mesh=plsc.VectorSubcoreMesh(core_axis_name="c", subcore_axis_name="s"),
        scratch_types=[pltpu.VMEM((CH,), jnp.int32),
                       pltpu.VMEM((CH, D), jnp.float32),
                       pltpu.VMEM((CH,), jnp.int32),
                       pltpu.VMEM((CH, D), jnp.float32),
                       pltpu.SemaphoreType.DMA,
                       pltpu.SemaphoreType.DMA],
    )


def _gather(h, dst, src):
    return _build_gather(dst.shape[0])(h, dst, src)


def _scatter_body(msg_hbm, dst_hbm, zeros_hbm, out_hbm, idx_v, rows_v, acc_sh):
    c = lax.axis_index("c")
    s = lax.axis_index("s")
    wid = _worker_id()
    nk = _num_chunks(wid, dst_hbm.shape[0] // CH)

    pltpu.sync_copy(zeros_hbm.at[pl.ds(s * RPT, RPT)],
                    acc_sh.at[pl.ds(s * RPT, RPT)])
    plsc.subcore_barrier()

    def step(j, carry):
        off = (j * NW + wid) * CH
        pltpu.sync_copy(msg_hbm.at[pl.ds(off, CH)], rows_v)
        pltpu.sync_copy(dst_hbm.at[pl.ds(off, CH)], idx_v)
        pltpu.sync_copy(rows_v, acc_sh.at[idx_v], add=True)
        return carry

    lax.fori_loop(0, nk, step, 0)
    plsc.subcore_barrier()
    pltpu.sync_copy(acc_sh.at[pl.ds(s * RPT, RPT)],
                    out_hbm.at[c, pl.ds(s * RPT, RPT)])


@functools.lru_cache(maxsize=None)
def _build_scatter(ne):
    return pl.kernel(
        _scatter_body,
        out_type=[jax.ShapeDtypeStruct((NC, NPAD, D), jnp.float32)],
        mesh=plsc.VectorSubcoreMesh(core_axis_name="c", subcore_axis_name="s"),
        scratch_types=[pltpu.VMEM((CH,), jnp.int32),
                       pltpu.VMEM((CH, D), jnp.float32),
                       pltpu.VMEM_SHARED((NPAD, D), jnp.float32)],
    )


def _scatter(msg, dst, zeros_pad):
    return _build_scatter(dst.shape[0])(msg, dst, zeros_pad)


BE = 3200


def _mlp_body(hd_ref, hs_ref, ea_ref, ax_ref, W1_ref, b1_ref, W2_ref, b2_ref,
              out_ref):
    x = jnp.concatenate(
        [hd_ref[...].astype(jnp.bfloat16), hs_ref[...].astype(jnp.bfloat16),
         ea_ref[...].astype(jnp.bfloat16), ax_ref[...].astype(jnp.bfloat16)],
        axis=1)
    acc = jnp.dot(x, W1_ref[...], preferred_element_type=jnp.float32)
    acc += b1_ref[...]
    hdn = jnp.maximum(acc, 0.0).astype(jnp.bfloat16)
    out_ref[...] = jnp.dot(hdn, W2_ref[...],
                           preferred_element_type=jnp.float32) + b2_ref[...]


def _edge_mlp(hd, hs, ea, ax, W1r, b1r, W2r, b2r, blk_off):
    n_in = 2 * D + ea.shape[1] + ax.shape[1]
    hid = W1r.shape[1]
    ne = hd.shape[0]
    grid = ne // BE
    shifted = lambda i, o=blk_off: (i + o, 0)
    local = lambda i: (i, 0)
    return pl.pallas_call(
        _mlp_body,
        grid=(grid,),
        in_specs=[
            pl.BlockSpec((BE, D), local),
            pl.BlockSpec((BE, D), local),
            pl.BlockSpec((BE, ea.shape[1]), shifted),
            pl.BlockSpec((BE, ax.shape[1]), shifted),
            pl.BlockSpec((n_in, hid), lambda i: (0, 0)),
            pl.BlockSpec((1, hid), lambda i: (0, 0)),
            pl.BlockSpec((hid, D), lambda i: (0, 0)),
            pl.BlockSpec((1, D), lambda i: (0, 0)),
        ],
        out_specs=pl.BlockSpec((BE, D), local),
        out_shape=jax.ShapeDtypeStruct((ne, D), jnp.float32),
    )(hd, hs, ea, ax, W1r, b1r, W2r, b2r)


BN = 2000


def _gru_body(*refs):
    ap_refs = refs[:KSLICE]
    h_ref, Wih_ref, bih_ref, Whh_ref, bhh_ref, out_ref = refs[KSLICE:]
    a = ap_refs[0][0] + ap_refs[0][1]
    for apr in ap_refs[1:]:
        a = a + apr[0] + apr[1]
    h = h_ref[...]
    gi = lax.dot_general(a, Wih_ref[...], (((1,), (1,)), ((), ())),
                         preferred_element_type=jnp.float32) + bih_ref[...]
    gh = lax.dot_general(h, Whh_ref[...], (((1,), (1,)), ((), ())),
                         preferred_element_type=jnp.float32) + bhh_ref[...]
    r = jax.nn.sigmoid(gi[:, 0:D] + gh[:, 0:D])
    z = jax.nn.sigmoid(gi[:, D:2 * D] + gh[:, D:2 * D])
    n = jnp.tanh(gi[:, 2 * D:3 * D] + r * gh[:, 2 * D:3 * D])
    out_ref[...] = (1.0 - z) * n + z * h


def _gru(aps, h, Wihr, bihr, Whhr, bhhr):
    grid = N_NODES // BN
    ap_spec = pl.BlockSpec((NC, BN, D), lambda i: (0, i, 0))
    return pl.pallas_call(
        _gru_body,
        grid=(grid,),
        in_specs=[ap_spec] * len(aps) + [
            pl.BlockSpec((BN, D), lambda i: (i, 0)),
            pl.BlockSpec((3 * D, D), lambda i: (0, 0)),
            pl.BlockSpec((1, 3 * D), lambda i: (0, 0)),
            pl.BlockSpec((3 * D, D), lambda i: (0, 0)),
            pl.BlockSpec((1, 3 * D), lambda i: (0, 0)),
        ],
        out_specs=pl.BlockSpec((BN, D), lambda i: (i, 0)),
        out_shape=jax.ShapeDtypeStruct((N_NODES, D), jnp.float32),
    )(*aps, h, Wihr, bihr, Whhr, bhhr)


def kernel(x, edge_index, edge_attr, auxiliary, W1, b1, W2, b2,
           W_ih, b_ih, W_hh, b_hh):
    ei = edge_index.astype(jnp.int32)
    src = ei[0]
    dst = ei[1]
    dst_s = [lax.slice(dst, (k * NE_S,), ((k + 1) * NE_S,))
             for k in range(KSLICE)]
    src_s = [lax.slice(src, (k * NE_S,), ((k + 1) * NE_S,))
             for k in range(KSLICE)]
    zeros_pad = jnp.zeros((NPAD, D), jnp.float32)
    W1_bf = W1.astype(jnp.bfloat16)
    W2_bf = W2.astype(jnp.bfloat16)
    h = x
    for r in range(W1.shape[0]):
        aps = []
        for k in range(KSLICE):
            hd, hs = _gather(h, dst_s[k], src_s[k])
            msg = _edge_mlp(hd, hs, edge_attr, auxiliary,
                            W1_bf[r], b1[r].reshape(1, -1),
                            W2_bf[r], b2[r].reshape(1, -1),
                            k * (NE_S // BE))
            (ap,) = _scatter(msg, dst_s[k], zeros_pad)
            aps.append(ap)
        h = _gru(aps, h, W_ih[r], b_ih[r].reshape(1, -1),
                 W_hh[r], b_hh[r].reshape(1, -1))
    return h


# speedup vs baseline: 3.4736x; 1.0163x over previous
import functools

import jax
import jax.numpy as jnp
from jax import lax
from jax.experimental import pallas as pl
from jax.experimental.pallas import tpu as pltpu
from jax.experimental.pallas import tpu_sc as plsc

N_NODES = 10000
N_EDGES = 160000
D = 128
NPAD = 10240

NC, NS = 2, 16
NW = NC * NS
CH = 128
RPT = NPAD // NS

KSLICE = 5
NE_S = N_EDGES // KSLICE


def _worker_id():
    return lax.axis_index("s") * NC + lax.axis_index("c")


def _num_chunks(wid, nchunk):
    base = nchunk // NW
    rem = nchunk - base * NW
    return base + (wid < rem).astype(jnp.int32)


def _gather_body(h_hbm, dst_hbm, src_hbm, hd_hbm, hs_hbm,
                 idx_d, rows_d, idx_s, rows_s, sem_d, sem_s):
    wid = _worker_id()
    nk = _num_chunks(wid, dst_hbm.shape[0] // CH)

    def step(j, carry):
        off = (j * NW + wid) * CH
        pltpu.sync_copy(dst_hbm.at[pl.ds(off, CH)], idx_d)
        cp_d = pltpu.async_copy(h_hbm.at[idx_d], rows_d, sem_d)
        pltpu.sync_copy(src_hbm.at[pl.ds(off, CH)], idx_s)
        cp_s = pltpu.async_copy(h_hbm.at[idx_s], rows_s, sem_s)
        cp_d.wait()
        pltpu.sync_copy(rows_d, hd_hbm.at[pl.ds(off, CH)])
        cp_s.wait()
        pltpu.sync_copy(rows_s, hs_hbm.at[pl.ds(off, CH)])
        return carry

    lax.fori_loop(0, nk, step, 0)


@functools.lru_cache(maxsize=None)
def _build_gather(ne):
    return pl.kernel(
        _gather_body,
        out_type=[jax.ShapeDtypeStruct((ne, D), jnp.float32),
                  jax.ShapeDtypeStruct((ne, D), jnp.float32)],
        mesh=plsc.VectorSubcoreMesh(core_axis_name="c", subcore_axis_name="s"),
        scratch_types=[pltpu.VMEM((CH,), jnp.int32),
                       pltpu.VMEM((CH, D), jnp.float32),
                       pltpu.VMEM((CH,), jnp.int32),
                       pltpu.VMEM((CH, D), jnp.float32),
                       pltpu.SemaphoreType.DMA,
                       pltpu.SemaphoreType.DMA],
    )


def _gather(h, dst, src):
    return _build_gather(dst.shape[0])(h, dst, src)


def _scatter_body(msg_hbm, dst_hbm, zeros_hbm, out_hbm, idx_v, rows_v, acc_sh):
    c = lax.axis_index("c")
    s = lax.axis_index("s")
    wid = _worker_id()
    nk = _num_chunks(wid, dst_hbm.shape[0] // CH)

    pltpu.sync_copy(zeros_hbm.at[pl.ds(s * RPT, RPT)],
                    acc_sh.at[pl.ds(s * RPT, RPT)])
    plsc.subcore_barrier()

    def step(j, carry):
        off = (j * NW + wid) * CH
        pltpu.sync_copy(msg_hbm.at[pl.ds(off, CH)], rows_v)
        pltpu.sync_copy(dst_hbm.at[pl.ds(off, CH)], idx_v)
        pltpu.sync_copy(rows_v, acc_sh.at[idx_v], add=True)
        return carry

    lax.fori_loop(0, nk, step, 0)
    plsc.subcore_barrier()
    pltpu.sync_copy(acc_sh.at[pl.ds(s * RPT, RPT)],
                    out_hbm.at[c, pl.ds(s * RPT, RPT)])


@functools.lru_cache(maxsize=None)
def _build_scatter(ne):
    return pl.kernel(
        _scatter_body,
        out_type=[jax.ShapeDtypeStruct((NC, NPAD, D), jnp.float32)],
        mesh=plsc.VectorSubcoreMesh(core_axis_name="c", subcore_axis_name="s"),
        scratch_types=[pltpu.VMEM((CH,), jnp.int32),
                       pltpu.VMEM((CH, D), jnp.float32),
                       pltpu.VMEM_SHARED((NPAD, D), jnp.float32)],
    )


def _scatter(msg, dst, zeros_pad):
    return _build_scatter(dst.shape[0])(msg, dst, zeros_pad)


BE = 3200


def _mlp_body(hd_ref, hs_ref, ea_ref, ax_ref, W1_ref, b1_ref, W2_ref, b2_ref,
              out_ref):
    x = jnp.concatenate(
        [hd_ref[...].astype(jnp.bfloat16), hs_ref[...].astype(jnp.bfloat16),
         ea_ref[...].astype(jnp.bfloat16), ax_ref[...]], axis=1)
    acc = jnp.dot(x, W1_ref[...], preferred_element_type=jnp.float32)
    acc += b1_ref[...]
    hdn = jnp.maximum(acc, 0.0).astype(jnp.bfloat16)
    out_ref[...] = jnp.dot(hdn, W2_ref[...],
                           preferred_element_type=jnp.float32) + b2_ref[...]


def _edge_mlp(hd, hs, ea, ax, W1r, b1r, W2r, b2r, blk_off):
    n_in = 2 * D + ea.shape[1] + ax.shape[1]
    hid = W1r.shape[1]
    ne = hd.shape[0]
    grid = ne // BE
    shifted = lambda i, o=blk_off: (i + o, 0)
    local = lambda i: (i, 0)
    return pl.pallas_call(
        _mlp_body,
        grid=(grid,),
        in_specs=[
            pl.BlockSpec((BE, D), local),
            pl.BlockSpec((BE, D), local),
            pl.BlockSpec((BE, ea.shape[1]), shifted),
            pl.BlockSpec((BE, ax.shape[1]), shifted),
            pl.BlockSpec((n_in, hid), lambda i: (0, 0)),
            pl.BlockSpec((1, hid), lambda i: (0, 0)),
            pl.BlockSpec((hid, D), lambda i: (0, 0)),
            pl.BlockSpec((1, D), lambda i: (0, 0)),
        ],
        out_specs=pl.BlockSpec((BE, D), local),
        out_shape=jax.ShapeDtypeStruct((ne, D), jnp.float32),
    )(hd, hs, ea, ax, W1r, b1r, W2r, b2r)


BN = 2000


def _gru_body(*refs):
    ap_refs = refs[:KSLICE]
    h_ref, Wih_ref, bih_ref, Whh_ref, bhh_ref, out_ref = refs[KSLICE:]
    a = ap_refs[0][0] + ap_refs[0][1]
    for apr in ap_refs[1:]:
        a = a + apr[0] + apr[1]
    h = h_ref[...]
    gi = lax.dot_general(a, Wih_ref[...], (((1,), (1,)), ((), ())),
                         preferred_element_type=jnp.float32) + bih_ref[...]
    gh = lax.dot_general(h, Whh_ref[...], (((1,), (1,)), ((), ())),
                         preferred_element_type=jnp.float32) + bhh_ref[...]
    r = jax.nn.sigmoid(gi[:, 0:D] + gh[:, 0:D])
    z = jax.nn.sigmoid(gi[:, D:2 * D] + gh[:, D:2 * D])
    n = jnp.tanh(gi[:, 2 * D:3 * D] + r * gh[:, 2 * D:3 * D])
    out_ref[...] = (1.0 - z) * n + z * h


def _gru(aps, h, Wihr, bihr, Whhr, bhhr):
    grid = N_NODES // BN
    ap_spec = pl.BlockSpec((NC, BN, D), lambda i: (0, i, 0))
    return pl.pallas_call(
        _gru_body,
        grid=(grid,),
        in_specs=[ap_spec] * len(aps) + [
            pl.BlockSpec((BN, D), lambda i: (i, 0)),
            pl.BlockSpec((3 * D, D), lambda i: (0, 0)),
            pl.BlockSpec((1, 3 * D), lambda i: (0, 0)),
            pl.BlockSpec((3 * D, D), lambda i: (0, 0)),
            pl.BlockSpec((1, 3 * D), lambda i: (0, 0)),
        ],
        out_specs=pl.BlockSpec((BN, D), lambda i: (i, 0)),
        out_shape=jax.ShapeDtypeStruct((N_NODES, D), jnp.float32),
    )(*aps, h, Wihr, bihr, Whhr, bhhr)


def kernel(x, edge_index, edge_attr, auxiliary, W1, b1, W2, b2,
           W_ih, b_ih, W_hh, b_hh):
    ei = edge_index.astype(jnp.int32)
    src = ei[0]
    dst = ei[1]
    dst_s = [lax.slice(dst, (k * NE_S,), ((k + 1) * NE_S,))
             for k in range(KSLICE)]
    src_s = [lax.slice(src, (k * NE_S,), ((k + 1) * NE_S,))
             for k in range(KSLICE)]
    zeros_pad = jnp.zeros((NPAD, D), jnp.float32)
    ax_bf = auxiliary.astype(jnp.bfloat16)
    W1_bf = W1.astype(jnp.bfloat16)
    W2_bf = W2.astype(jnp.bfloat16)
    h = x
    for r in range(W1.shape[0]):
        aps = []
        for k in range(KSLICE):
            hd, hs = _gather(h, dst_s[k], src_s[k])
            msg = _edge_mlp(hd, hs, edge_attr, ax_bf,
                            W1_bf[r], b1[r].reshape(1, -1),
                            W2_bf[r], b2[r].reshape(1, -1),
                            k * (NE_S // BE))
            (ap,) = _scatter(msg, dst_s[k], zeros_pad)
            aps.append(ap)
        h = _gru(aps, h, W_ih[r], b_ih[r].reshape(1, -1),
                 W_hh[r], b_hh[r].reshape(1, -1))
    return h


# speedup vs baseline: 3.5034x; 1.0086x over previous
import functools

import jax
import jax.numpy as jnp
from jax import lax
from jax.experimental import pallas as pl
from jax.experimental.pallas import tpu as pltpu
from jax.experimental.pallas import tpu_sc as plsc

N_NODES = 10000
N_EDGES = 160000
D = 128
NPAD = 10240

NC, NS = 2, 16
NW = NC * NS
CH = 128
RPT = NPAD // NS

KSLICE = 2
NE_S = N_EDGES // KSLICE


def _worker_id():
    return lax.axis_index("s") * NC + lax.axis_index("c")


def _num_chunks(wid, nchunk):
    base = nchunk // NW
    rem = nchunk - base * NW
    return base + (wid < rem).astype(jnp.int32)


def _gather_body(h_hbm, dst_hbm, src_hbm, hd_hbm, hs_hbm,
                 idx_d, rows_d, idx_s, rows_s, sem_d, sem_s):
    wid = _worker_id()
    nk = _num_chunks(wid, dst_hbm.shape[0] // CH)

    def step(j, carry):
        off = (j * NW + wid) * CH
        pltpu.sync_copy(dst_hbm.at[pl.ds(off, CH)], idx_d)
        cp_d = pltpu.async_copy(h_hbm.at[idx_d], rows_d, sem_d)
        pltpu.sync_copy(src_hbm.at[pl.ds(off, CH)], idx_s)
        cp_s = pltpu.async_copy(h_hbm.at[idx_s], rows_s, sem_s)
        cp_d.wait()
        pltpu.sync_copy(rows_d, hd_hbm.at[pl.ds(off, CH)])
        cp_s.wait()
        pltpu.sync_copy(rows_s, hs_hbm.at[pl.ds(off, CH)])
        return carry

    lax.fori_loop(0, nk, step, 0)


@functools.lru_cache(maxsize=None)
def _build_gather(ne):
    return pl.kernel(
        _gather_body,
        out_type=[jax.ShapeDtypeStruct((ne, D), jnp.float32),
                  jax.ShapeDtypeStruct((ne, D), jnp.float32)],
        mesh=plsc.VectorSubcoreMesh(core_axis_name="c", subcore_axis_name="s"),
        scratch_types=[pltpu.VMEM((CH,), jnp.int32),
                       pltpu.VMEM((CH, D), jnp.float32),
                       pltpu.VMEM((CH,), jnp.int32),
                       pltpu.VMEM((CH, D), jnp.float32),
                       pltpu.SemaphoreType.DMA,
                       pltpu.SemaphoreType.DMA],
    )


def _gather(h, dst, src):
    return _build_gather(dst.shape[0])(h, dst, src)


def _scatter_body(msg_hbm, dst_hbm, zeros_hbm, out_hbm, idx_v, rows_v, acc_sh):
    c = lax.axis_index("c")
    s = lax.axis_index("s")
    wid = _worker_id()
    nk = _num_chunks(wid, dst_hbm.shape[0] // CH)

    pltpu.sync_copy(zeros_hbm.at[pl.ds(s * RPT, RPT)],
                    acc_sh.at[pl.ds(s * RPT, RPT)])
    plsc.subcore_barrier()

    def step(j, carry):
        off = (j * NW + wid) * CH
        pltpu.sync_copy(msg_hbm.at[pl.ds(off, CH)], rows_v)
        pltpu.sync_copy(dst_hbm.at[pl.ds(off, CH)], idx_v)
        pltpu.sync_copy(rows_v, acc_sh.at[idx_v], add=True)
        return carry

    lax.fori_loop(0, nk, step, 0)
    plsc.subcore_barrier()
    pltpu.sync_copy(acc_sh.at[pl.ds(s * RPT, RPT)],
                    out_hbm.at[c, pl.ds(s * RPT, RPT)])


@functools.lru_cache(maxsize=None)
def _build_scatter(ne):
    return pl.kernel(
        _scatter_body,
        out_type=[jax.ShapeDtypeStruct((NC, NPAD, D), jnp.float32)],
        mesh=plsc.VectorSubcoreMesh(core_axis_name="c", subcore_axis_name="s"),
        scratch_types=[pltpu.VMEM((CH,), jnp.int32),
                       pltpu.VMEM((CH, D), jnp.float32),
                       pltpu.VMEM_SHARED((NPAD, D), jnp.float32)],
    )


def _scatter(msg, dst, zeros_pad):
    return _build_scatter(dst.shape[0])(msg, dst, zeros_pad)


BE = 3200


def _mlp_body(hd_ref, hs_ref, ea_ref, ax_ref, W1_ref, b1_ref, W2_ref, b2_ref,
              out_ref):
    x = jnp.concatenate(
        [hd_ref[...].astype(jnp.bfloat16), hs_ref[...].astype(jnp.bfloat16),
         ea_ref[...].astype(jnp.bfloat16), ax_ref[...]], axis=1)
    acc = jnp.dot(x, W1_ref[...], preferred_element_type=jnp.float32)
    acc += b1_ref[...]
    hdn = jnp.maximum(acc, 0.0).astype(jnp.bfloat16)
    out_ref[...] = jnp.dot(hdn, W2_ref[...],
                           preferred_element_type=jnp.float32) + b2_ref[...]


def _edge_mlp(hd, hs, ea, ax, W1r, b1r, W2r, b2r, blk_off):
    n_in = 2 * D + ea.shape[1] + ax.shape[1]
    hid = W1r.shape[1]
    ne = hd.shape[0]
    grid = ne // BE
    shifted = lambda i, o=blk_off: (i + o, 0)
    local = lambda i: (i, 0)
    return pl.pallas_call(
        _mlp_body,
        grid=(grid,),
        in_specs=[
            pl.BlockSpec((BE, D), local),
            pl.BlockSpec((BE, D), local),
            pl.BlockSpec((BE, ea.shape[1]), shifted),
            pl.BlockSpec((BE, ax.shape[1]), shifted),
            pl.BlockSpec((n_in, hid), lambda i: (0, 0)),
            pl.BlockSpec((1, hid), lambda i: (0, 0)),
            pl.BlockSpec((hid, D), lambda i: (0, 0)),
            pl.BlockSpec((1, D), lambda i: (0, 0)),
        ],
        out_specs=pl.BlockSpec((BE, D), local),
        out_shape=jax.ShapeDtypeStruct((ne, D), jnp.float32),
    )(hd, hs, ea, ax, W1r, b1r, W2r, b2r)


BN = 2000


def _gru_body(*refs):
    ap_refs = refs[:KSLICE]
    h_ref, Wih_ref, bih_ref, Whh_ref, bhh_ref, out_ref = refs[KSLICE:]
    a = ap_refs[0][0] + ap_refs[0][1]
    for apr in ap_refs[1:]:
        a = a + apr[0] + apr[1]
    h = h_ref[...]
    gi = lax.dot_general(a, Wih_ref[...], (((1,), (1,)), ((), ())),
                         preferred_element_type=jnp.float32) + bih_ref[...]
    gh = lax.dot_general(h, Whh_ref[...], (((1,), (1,)), ((), ())),
                         preferred_element_type=jnp.float32) + bhh_ref[...]
    r = jax.nn.sigmoid(gi[:, 0:D] + gh[:, 0:D])
    z = jax.nn.sigmoid(gi[:, D:2 * D] + gh[:, D:2 * D])
    n = jnp.tanh(gi[:, 2 * D:3 * D] + r * gh[:, 2 * D:3 * D])
    out_ref[...] = (1.0 - z) * n + z * h


def _gru(aps, h, Wihr, bihr, Whhr, bhhr):
    grid = N_NODES // BN
    ap_spec = pl.BlockSpec((NC, BN, D), lambda i: (0, i, 0))
    return pl.pallas_call(
        _gru_body,
        grid=(grid,),
        in_specs=[ap_spec] * len(aps) + [
            pl.BlockSpec((BN, D), lambda i: (i, 0)),
            pl.BlockSpec((3 * D, D), lambda i: (0, 0)),
            pl.BlockSpec((1, 3 * D), lambda i: (0, 0)),
            pl.BlockSpec((3 * D, D), lambda i: (0, 0)),
            pl.BlockSpec((1, 3 * D), lambda i: (0, 0)),
        ],
        out_specs=pl.BlockSpec((BN, D), lambda i: (i, 0)),
        out_shape=jax.ShapeDtypeStruct((N_NODES, D), jnp.float32),
    )(*aps, h, Wihr, bihr, Whhr, bhhr)


def kernel(x, edge_index, edge_attr, auxiliary, W1, b1, W2, b2,
           W_ih, b_ih, W_hh, b_hh):
    ei = edge_index.astype(jnp.int32)
    src = ei[0]
    dst = ei[1]
    dst_s = [lax.slice(dst, (k * NE_S,), ((k + 1) * NE_S,))
             for k in range(KSLICE)]
    src_s = [lax.slice(src, (k * NE_S,), ((k + 1) * NE_S,))
             for k in range(KSLICE)]
    zeros_pad = jnp.zeros((NPAD, D), jnp.float32)
    ax_bf = auxiliary.astype(jnp.bfloat16)
    W1_bf = W1.astype(jnp.bfloat16)
    W2_bf = W2.astype(jnp.bfloat16)
    h = x
    for r in range(W1.shape[0]):
        aps = []
        for k in range(KSLICE):
            hd, hs = _gather(h, dst_s[k], src_s[k])
            msg = _edge_mlp(hd, hs, edge_attr, ax_bf,
                            W1_bf[r], b1[r].reshape(1, -1),
                            W2_bf[r], b2[r].reshape(1, -1),
                            k * (NE_S // BE))
            (ap,) = _scatter(msg, dst_s[k], zeros_pad)
            aps.append(ap)
        h = _gru(aps, h, W_ih[r], b_ih[r].reshape(1, -1),
                 W_hh[r], b_hh[r].reshape(1, -1))
    return h
